# baseline (speedup 1.0000x reference)
import functools

import jax
import jax.numpy as jnp
from jax import lax
from jax.experimental import pallas as pl
from jax.experimental.pallas import tpu as pltpu
from jax.experimental.pallas import tpu_sc as plsc

_B, _D, _T = 8, 512, 1024
_NQ, _K, _CD = 4, 8192, 256
_BT = _B * _T

_TB = 512
_KB = 512
_KW = 2048


def _prep(in_v, in_g, out_v, out_g, codebooks):
    ni = jnp.sqrt(jnp.sum(in_v * in_v, axis=2, keepdims=True))
    wi = in_g[:, :, None] * in_v / ni
    no = jnp.sqrt(jnp.sum(out_v * out_v, axis=2, keepdims=True))
    wo = out_g[:, :, None] * out_v / no
    cn = jnp.sqrt(jnp.sum(codebooks * codebooks, axis=2, keepdims=True))
    cbn = codebooks / jnp.maximum(cn, 1e-12)
    ss = jnp.sum(cbn * cbn, axis=2, keepdims=True)
    return wi, wo, cbn.astype(jnp.bfloat16), ss


def _encode_body(z_ref, wi_ref, ib_ref, cbn_ref, ss_ref, zi_ref, idx_ref):
    z = z_ref[...]
    zi = jnp.dot(wi_ref[...].astype(jnp.bfloat16), z.astype(jnp.bfloat16),
                 preferred_element_type=jnp.float32)
    zi = zi + ib_ref[...]
    zi_ref[...] = zi
    nrm = jnp.sqrt(jnp.sum(zi * zi, axis=0, keepdims=True))
    enc = zi / jnp.maximum(nrm, 1e-12)
    s1 = jnp.sum(enc * enc, axis=0, keepdims=True)
    encb = enc.astype(jnp.bfloat16)

    def blk(j, carry):
        g, bv, bi = carry
        i = g * (_KW // _KB) + j
        off = pl.multiple_of(i * _KB, _KB)
        cb = cbn_ref[pl.ds(off, _KB), :]
        ss = ss_ref[pl.ds(off, _KB), :]
        dot = jnp.dot(cb, encb, preferred_element_type=jnp.float32)
        s = -((s1 - 2.0 * dot) + ss)
        bm = jnp.max(s, axis=0, keepdims=True)
        rows = lax.broadcasted_iota(jnp.int32, (_KB, _TB), 0) + i * _KB
        cand = jnp.where(s == bm, rows, _K)
        bj = jnp.min(cand, axis=0, keepdims=True)
        upd = bm > bv
        return g, jnp.where(upd, bm, bv), jnp.where(upd, bj, bi)

    def group(g, carry):
        av, ai = carry
        gv0 = jnp.full((1, _TB), -jnp.inf, jnp.float32)
        gi0 = jnp.zeros((1, _TB), jnp.int32)
        _, gv, gi = lax.fori_loop(0, _KW // _KB, blk, (g, gv0, gi0))
        steal = gv > av
        av = jnp.where(steal, gv, av).astype(jnp.bfloat16).astype(jnp.float32)
        ai = jnp.where(steal, gi, ai)
        return av, ai

    av0 = jnp.full((1, _TB), -jnp.inf, jnp.float32)
    ai0 = jnp.zeros((1, _TB), jnp.int32)
    _, ai = lax.fori_loop(0, _K // _KW, group, (av0, ai0))
    del _
    idx_ref[...] = ai


def _encode(z_res, wi, ib, cbn, ss):
    return pl.pallas_call(
        _encode_body,
        grid=(_B, _T // _TB),
        in_specs=[
            pl.BlockSpec((None, _D, _TB), lambda b, t: (b, 0, t)),
            pl.BlockSpec((_CD, _D), lambda b, t: (0, 0)),
            pl.BlockSpec((_CD, 1), lambda b, t: (0, 0)),
            pl.BlockSpec((_K, _CD), lambda b, t: (0, 0)),
            pl.BlockSpec((_K, 1), lambda b, t: (0, 0)),
        ],
        out_specs=[
            pl.BlockSpec((None, _CD, _TB), lambda b, t: (b, 0, t)),
            pl.BlockSpec((None, 1, _TB), lambda b, t: (b, 0, t)),
        ],
        out_shape=[
            jax.ShapeDtypeStruct((_B, _CD, _T), jnp.float32),
            jax.ShapeDtypeStruct((_B, 1, _T), jnp.int32),
        ],
        compiler_params=pltpu.CompilerParams(
            dimension_semantics=("parallel", "parallel")),
    )(z_res, wi, ib, cbn, ss)


_BPW = _BT // 32


@functools.lru_cache(maxsize=None)
def _make_sc_gather():
    mesh = plsc.VectorSubcoreMesh(core_axis_name="c", subcore_axis_name="s")
    nc = mesh.num_cores

    @functools.partial(
        pl.kernel,
        mesh=mesh,
        out_type=jax.ShapeDtypeStruct((_BT, _CD), jnp.float32),
        scratch_types=[
            pltpu.VMEM((_BPW,), jnp.int32),
            pltpu.VMEM((_BPW, _CD), jnp.float32),
            pltpu.SemaphoreType.DMA,
        ],
    )
    def _sc_gather(table_hbm, idx_hbm, out_hbm, idx_v, rows_v, sem):
        wid = lax.axis_index("s") * nc + lax.axis_index("c")
        base = wid * _BPW
        pltpu.sync_copy(idx_hbm.at[pl.ds(base, _BPW)], idx_v)
        pltpu.async_copy(table_hbm.at[idx_v], rows_v, sem).wait()
        pltpu.sync_copy(rows_v, out_hbm.at[pl.ds(base, _BPW)])

    return _sc_gather


def _decode_body(zq_ref, wo_ref, ob_ref, res_ref, zi_ref, zqt_ref, zo_ref,
                 rout_ref):
    zit = zi_ref[...]
    zqt = zit + (zq_ref[...].T - zit)
    zqt_ref[...] = zqt
    zo = jnp.dot(wo_ref[...].astype(jnp.bfloat16), zqt.astype(jnp.bfloat16),
                 preferred_element_type=jnp.float32)
    zo = zo + ob_ref[...]
    zo_ref[...] = zo
    rout_ref[...] = res_ref[...] - zo


def _decode(zq_rows, wo, ob, res_in, zi):
    return pl.pallas_call(
        _decode_body,
        grid=(_B, _T // _TB),
        in_specs=[
            pl.BlockSpec((None, _TB, _CD), lambda b, t: (b, t, 0)),
            pl.BlockSpec((_D, _CD), lambda b, t: (0, 0)),
            pl.BlockSpec((_D, 1), lambda b, t: (0, 0)),
            pl.BlockSpec((None, _D, _TB), lambda b, t: (b, 0, t)),
            pl.BlockSpec((None, _CD, _TB), lambda b, t: (b, 0, t)),
        ],
        out_specs=[
            pl.BlockSpec((None, _CD, _TB), lambda b, t: (b, 0, t)),
            pl.BlockSpec((None, _D, _TB), lambda b, t: (b, 0, t)),
            pl.BlockSpec((None, _D, _TB), lambda b, t: (b, 0, t)),
        ],
        out_shape=[
            jax.ShapeDtypeStruct((_B, _CD, _T), jnp.float32),
            jax.ShapeDtypeStruct((_B, _D, _T), jnp.float32),
            jax.ShapeDtypeStruct((_B, _D, _T), jnp.float32),
        ],
        compiler_params=pltpu.CompilerParams(
            dimension_semantics=("parallel", "parallel")),
    )(zq_rows, wo, ob, res_in, zi)


def _decode_last_body(zq_ref, wo_ref, ob_ref, res_ref, z_ref, zi_ref,
                      zqt_ref, zo_ref, rout_ref, zO_ref):
    zit = zi_ref[...]
    zqt = zit + (zq_ref[...].T - zit)
    zqt_ref[...] = zqt
    zo = jnp.dot(wo_ref[...].astype(jnp.bfloat16), zqt.astype(jnp.bfloat16),
                 preferred_element_type=jnp.float32)
    zo = zo + ob_ref[...]
    zo_ref[...] = zo
    rout = res_ref[...] - zo
    rout_ref[...] = rout
    zO_ref[...] = z_ref[...] - rout


def _decode_last(zq_rows, wo, ob, res_in, z, zi):
    return pl.pallas_call(
        _decode_last_body,
        grid=(_B, _T // _TB),
        in_specs=[
            pl.BlockSpec((None, _TB, _CD), lambda b, t: (b, t, 0)),
            pl.BlockSpec((_D, _CD), lambda b, t: (0, 0)),
            pl.BlockSpec((_D, 1), lambda b, t: (0, 0)),
            pl.BlockSpec((None, _D, _TB), lambda b, t: (b, 0, t)),
            pl.BlockSpec((None, _D, _TB), lambda b, t: (b, 0, t)),
            pl.BlockSpec((None, _CD, _TB), lambda b, t: (b, 0, t)),
        ],
        out_specs=[
            pl.BlockSpec((None, _CD, _TB), lambda b, t: (b, 0, t)),
            pl.BlockSpec((None, _D, _TB), lambda b, t: (b, 0, t)),
            pl.BlockSpec((None, _D, _TB), lambda b, t: (b, 0, t)),
            pl.BlockSpec((None, _D, _TB), lambda b, t: (b, 0, t)),
        ],
        out_shape=[
            jax.ShapeDtypeStruct((_B, _CD, _T), jnp.float32),
            jax.ShapeDtypeStruct((_B, _D, _T), jnp.float32),
            jax.ShapeDtypeStruct((_B, _D, _T), jnp.float32),
            jax.ShapeDtypeStruct((_B, _D, _T), jnp.float32),
        ],
        compiler_params=pltpu.CompilerParams(
            dimension_semantics=("parallel", "parallel")),
    )(zq_rows, wo, ob, res_in, z, zi)


def kernel(z, in_v, in_g, in_b, out_v, out_g, out_b, codebooks):
    wi, wo, cbn, ss = _prep(in_v, in_g, out_v, out_g, codebooks)
    ib = in_b.reshape(_NQ, _CD, 1)
    ob = out_b.reshape(_NQ, _D, 1)

    residual = z
    zO = None
    codes, zis, zqs, zos = [], [], [], []
    for q in range(_NQ):
        zi, idx3 = _encode(residual, wi[q], ib[q], cbn[q], ss[q])
        idx = idx3.reshape(_BT)
        zq_rows = _make_sc_gather()(codebooks[q], idx)
        zq_rows = zq_rows.reshape(_B, _T, _CD)
        if q < _NQ - 1:
            zqt, zo, residual = _decode(zq_rows, wo[q], ob[q], residual, zi)
        else:
            zqt, zo, residual, zO = _decode_last(
                zq_rows, wo[q], ob[q], residual, z, zi)
        codes.append(idx3.reshape(_B, _T))
        zis.append(zi)
        zqs.append(zqt)
        zos.append(zo)

    return (jnp.stack(codes, axis=1), zO, jnp.stack(zis, axis=1),
            jnp.stack(zqs, axis=1), jnp.stack(zos, axis=1))


# speedup vs baseline: 1.1461x; 1.1461x over previous
import functools

import jax
import jax.numpy as jnp
from jax import lax
from jax.experimental import pallas as pl
from jax.experimental.pallas import tpu as pltpu
from jax.experimental.pallas import tpu_sc as plsc

_B, _D, _T = 8, 512, 1024
_NQ, _K, _CD = 4, 8192, 256
_BT = _B * _T

_TB = 512
_KB = 512
_KW = 2048


def _prep(in_v, in_g, out_v, out_g, codebooks):
    ni = jnp.sqrt(jnp.sum(in_v * in_v, axis=2, keepdims=True))
    wi = in_g[:, :, None] * in_v / ni
    no = jnp.sqrt(jnp.sum(out_v * out_v, axis=2, keepdims=True))
    wo = out_g[:, :, None] * out_v / no
    cn = jnp.sqrt(jnp.sum(codebooks * codebooks, axis=2, keepdims=True))
    cbn = codebooks / jnp.maximum(cn, 1e-12)
    ss = jnp.sum(cbn * cbn, axis=2, keepdims=True)
    return wi, wo, cbn.astype(jnp.bfloat16), ss


def _encode_body(z_ref, wi_ref, ib_ref, cbn_ref, ss_ref, *rest):
    zi_ref, idx_ref = rest[-2], rest[-1]
    z = z_ref[...]
    zi = jnp.dot(wi_ref[...].astype(jnp.bfloat16), z.astype(jnp.bfloat16),
                 preferred_element_type=jnp.float32)
    zi = zi + ib_ref[...]
    zi_ref[...] = zi
    nrm = jnp.sqrt(jnp.sum(zi * zi, axis=0, keepdims=True))
    enc = zi / jnp.maximum(nrm, 1e-12)
    s1 = jnp.sum(enc * enc, axis=0, keepdims=True)
    encb = enc.astype(jnp.bfloat16)

    def blk(j, carry):
        g, bv, bi = carry
        i = g * (_KW // _KB) + j
        off = pl.multiple_of(i * _KB, _KB)
        cb = cbn_ref[pl.ds(off, _KB), :]
        ss = ss_ref[pl.ds(off, _KB), :]
        dot = jnp.dot(cb, encb, preferred_element_type=jnp.float32)
        s = -((s1 - 2.0 * dot) + ss)
        bm = jnp.max(s, axis=0, keepdims=True)
        rows = lax.broadcasted_iota(jnp.int32, (_KB, _TB), 0) + i * _KB
        cand = jnp.where(s == bm, rows, _K)
        bj = jnp.min(cand, axis=0, keepdims=True)
        upd = bm > bv
        return g, jnp.where(upd, bm, bv), jnp.where(upd, bj, bi)

    def group(g, carry):
        av, ai = carry
        gv0 = jnp.full((1, _TB), -jnp.inf, jnp.float32)
        gi0 = jnp.zeros((1, _TB), jnp.int32)
        _, gv, gi = lax.fori_loop(0, _KW // _KB, blk, (g, gv0, gi0))
        steal = gv > av
        av = jnp.where(steal, gv, av).astype(jnp.bfloat16).astype(jnp.float32)
        ai = jnp.where(steal, gi, ai)
        return av, ai

    av0 = jnp.full((1, _TB), -jnp.inf, jnp.float32)
    ai0 = jnp.zeros((1, _TB), jnp.int32)
    _, ai = lax.fori_loop(0, _K // _KW, group, (av0, ai0))
    del _
    idx_ref[...] = ai


def _encode(z_res, wi, ib, cbn, ss, q, zi_acc=None, idx_acc=None):
    zi_spec = pl.BlockSpec((None, None, _CD, _TB), lambda b, t: (b, q, 0, t))
    idx_spec = pl.BlockSpec((None, None, 1, _TB), lambda b, t: (b, q, 0, t))
    in_specs = [
        pl.BlockSpec((None, _D, _TB), lambda b, t: (b, 0, t)),
        pl.BlockSpec((_CD, _D), lambda b, t: (0, 0)),
        pl.BlockSpec((_CD, 1), lambda b, t: (0, 0)),
        pl.BlockSpec((_K, _CD), lambda b, t: (0, 0)),
        pl.BlockSpec((_K, 1), lambda b, t: (0, 0)),
    ]
    inputs = (z_res, wi, ib, cbn, ss)
    aliases = {}
    if q > 0:
        in_specs += [zi_spec, idx_spec]
        inputs += (zi_acc, idx_acc)
        aliases = {5: 0, 6: 1}
    return pl.pallas_call(
        _encode_body,
        grid=(_B, _T // _TB),
        in_specs=in_specs,
        out_specs=[zi_spec, idx_spec],
        out_shape=[
            jax.ShapeDtypeStruct((_B, _NQ, _CD, _T), jnp.float32),
            jax.ShapeDtypeStruct((_B, _NQ, 1, _T), jnp.int32),
        ],
        input_output_aliases=aliases,
        compiler_params=pltpu.CompilerParams(
            dimension_semantics=("parallel", "parallel")),
    )(*inputs)


_BPW = _BT // 32


@functools.lru_cache(maxsize=None)
def _make_sc_gather():
    mesh = plsc.VectorSubcoreMesh(core_axis_name="c", subcore_axis_name="s")
    nc = mesh.num_cores

    @functools.partial(
        pl.kernel,
        mesh=mesh,
        out_type=jax.ShapeDtypeStruct((_BT, _CD), jnp.float32),
        scratch_types=[
            pltpu.VMEM((_BPW,), jnp.int32),
            pltpu.VMEM((_BPW, _CD), jnp.float32),
            pltpu.SemaphoreType.DMA,
        ],
    )
    def _sc_gather(table_hbm, idx_hbm, out_hbm, idx_v, rows_v, sem):
        wid = lax.axis_index("s") * nc + lax.axis_index("c")
        base = wid * _BPW
        pltpu.sync_copy(idx_hbm.at[pl.ds(base, _BPW)], idx_v)
        pltpu.async_copy(table_hbm.at[idx_v], rows_v, sem).wait()
        pltpu.sync_copy(rows_v, out_hbm.at[pl.ds(base, _BPW)])

    return _sc_gather


def _decode_body(zq_ref, wo_ref, ob_ref, res_ref, zi_ref, *rest):
    last = len(rest) == 7
    if last:
        z_ref = rest[0]
        zqt_ref, zo_ref, rout_ref, zO_ref = rest[-4:]
    else:
        zqt_ref, zo_ref, rout_ref = rest[-3:]
    zit = zi_ref[...]
    zqt = zit + (zq_ref[...].T - zit)
    zqt_ref[...] = zqt
    zo = jnp.dot(wo_ref[...].astype(jnp.bfloat16), zqt.astype(jnp.bfloat16),
                 preferred_element_type=jnp.float32)
    zo = zo + ob_ref[...]
    zo_ref[...] = zo
    rout = res_ref[...] - zo
    rout_ref[...] = rout
    if last:
        zO_ref[...] = z_ref[...] - rout


def _decode(zq_rows, wo, ob, res_in, zi_acc, q, zq_acc=None, zo_acc=None,
            z=None):
    last = q == _NQ - 1
    zq_spec = pl.BlockSpec((None, None, _CD, _TB), lambda b, t: (b, q, 0, t))
    zo_spec = pl.BlockSpec((None, None, _D, _TB), lambda b, t: (b, q, 0, t))
    res_spec = pl.BlockSpec((None, _D, _TB), lambda b, t: (b, 0, t))
    in_specs = [
        pl.BlockSpec((None, _TB, _CD), lambda b, t: (b, t, 0)),
        pl.BlockSpec((_D, _CD), lambda b, t: (0, 0)),
        pl.BlockSpec((_D, 1), lambda b, t: (0, 0)),
        res_spec,
        pl.BlockSpec((None, None, _CD, _TB), lambda b, t: (b, q, 0, t)),
    ]
    inputs = (zq_rows, wo, ob, res_in, zi_acc)
    aliases = {}
    if last:
        in_specs.append(res_spec)
        inputs += (z,)
    if q > 0:
        in_specs += [zq_spec, zo_spec]
        inputs += (zq_acc, zo_acc)
        base = 6 if last else 5
        aliases = {base: 0, base + 1: 1}
    out_specs = [zq_spec, zo_spec, res_spec]
    out_shape = [
        jax.ShapeDtypeStruct((_B, _NQ, _CD, _T), jnp.float32),
        jax.ShapeDtypeStruct((_B, _NQ, _D, _T), jnp.float32),
        jax.ShapeDtypeStruct((_B, _D, _T), jnp.float32),
    ]
    if last:
        out_specs.append(res_spec)
        out_shape.append(jax.ShapeDtypeStruct((_B, _D, _T), jnp.float32))
    return pl.pallas_call(
        _decode_body,
        grid=(_B, _T // _TB),
        in_specs=in_specs,
        out_specs=out_specs,
        out_shape=out_shape,
        input_output_aliases=aliases,
        compiler_params=pltpu.CompilerParams(
            dimension_semantics=("parallel", "parallel")),
    )(*inputs)


def kernel(z, in_v, in_g, in_b, out_v, out_g, out_b, codebooks):
    wi, wo, cbn, ss = _prep(in_v, in_g, out_v, out_g, codebooks)
    ib = in_b.reshape(_NQ, _CD, 1)
    ob = out_b.reshape(_NQ, _D, 1)

    residual = z
    zO = None
    zi_acc = idx_acc = zq_acc = zo_acc = None
    for q in range(_NQ):
        zi_acc, idx_acc = _encode(residual, wi[q], ib[q], cbn[q], ss[q], q,
                                  zi_acc, idx_acc)
        idx = idx_acc[:, q].reshape(_BT)
        zq_rows = _make_sc_gather()(codebooks[q], idx)
        zq_rows = zq_rows.reshape(_B, _T, _CD)
        if q < _NQ - 1:
            zq_acc, zo_acc, residual = _decode(
                zq_rows, wo[q], ob[q], residual, zi_acc, q, zq_acc, zo_acc)
        else:
            zq_acc, zo_acc, residual, zO = _decode(
                zq_rows, wo[q], ob[q], residual, zi_acc, q, zq_acc, zo_acc,
                z)

    return (idx_acc.reshape(_B, _NQ, _T), zO, zi_acc, zq_acc, zo_acc)
